# baseline (speedup 1.0000x reference)
import functools

import jax
import jax.numpy as jnp
import numpy as np
from jax import lax
from jax.experimental import pallas as pl
from jax.experimental.pallas import tpu as pltpu
from jax.experimental.pallas import tpu_sc as plsc

N = 100000
C = 128
H = 224
NC, NS = 2, 16
NW = NC * NS
NPW = 3200
NPAD = NW * NPW
P = 64
NPW0 = 37 * P
NPW1 = 63 * P


def _prelude_body(xy_ref, idx_ref, w_ref):
    size = float(H)
    x = xy_ref[0:1, :]
    y = xy_ref[1:2, :]
    x = x - jnp.min(x)
    x = x * ((size - 0.01) / jnp.max(x))
    y = y - jnp.min(y)
    y = y * ((size - 0.01) / jnp.max(y))
    x1 = jnp.floor(x)
    y1 = jnp.floor(y)
    x2 = jnp.minimum(jnp.ceil(x), size - 1.0)
    y2 = jnp.minimum(jnp.ceil(y), size - 1.0)
    wx1 = jnp.where(x2 > x1, x2 - x, 0.0)
    wx2 = jnp.where(x2 > x1, x - x1, 0.0)
    wy1 = jnp.where(y2 > y1, y2 - y, 0.0)
    wy2 = jnp.where(y2 > y1, y - y1, 0.0)
    x1i = x1.astype(jnp.int32)
    x2i = x2.astype(jnp.int32)
    y1i = y1.astype(jnp.int32)
    y2i = y2.astype(jnp.int32)
    idx_ref[0:1, :] = x1i * H + y1i
    idx_ref[1:2, :] = x1i * H + y2i
    idx_ref[2:3, :] = x2i * H + y1i
    idx_ref[3:4, :] = x2i * H + y2i
    w_ref[0:1, :] = wx1 * wy1
    w_ref[1:2, :] = wx1 * wy2
    w_ref[2:3, :] = wx2 * wy1
    w_ref[3:4, :] = wx2 * wy2


def _prelude(xy_pad):
    return pl.pallas_call(
        _prelude_body,
        out_shape=(
            jax.ShapeDtypeStruct((4, NPAD), jnp.int32),
            jax.ShapeDtypeStruct((4, NPAD), jnp.float32),
        ),
    )(xy_pad)


def _sc_body(table_hbm, idx_hbm, w_hbm, out_hbm, idx_v, w_v, q_v,
             out_v, semi0, semi1, semq0, semq1, semo0, semo1):
    cidx = lax.axis_index("c")
    sidx = lax.axis_index("s")
    base = jnp.where(cidx == 0, sidx * NPW0, NS * NPW0 + sidx * NPW1)
    nit = jnp.where(cidx == 0, NPW0 // P, NPW1 // P)
    semi = (semi0, semi1)
    semq = (semq0, semq1)
    semo = (semo0, semo1)

    def idxw_start(it, b):
        off = base + it * P
        for k in range(4):
            pltpu.async_copy(idx_hbm.at[k, pl.ds(off, P)],
                             idx_v.at[b, k // 2, pl.ds((k % 2) * P, P)],
                             semi[b])
            pltpu.async_copy(w_hbm.at[k, pl.ds(off, P)], w_v.at[b, k],
                             semi[b])

    def idxw_wait(b):
        for k in range(4):
            pltpu.make_async_copy(idx_hbm.at[k, pl.ds(0, P)],
                                  idx_v.at[b, k // 2, pl.ds((k % 2) * P, P)],
                                  semi[b]).wait()
            pltpu.make_async_copy(w_hbm.at[k, pl.ds(0, P)], w_v.at[b, k],
                                  semi[b]).wait()

    def gather_start(b):
        for j in range(2):
            pltpu.async_copy(table_hbm.at[idx_v.at[b, j]], q_v.at[b, j],
                             semq[b])

    def gather_wait(b):
        for j in range(2):
            pltpu.make_async_copy(table_hbm.at[idx_v.at[b, j]], q_v.at[b, j],
                                  semq[b]).wait()

    def out_start(it, b):
        off = base + it * P
        pltpu.async_copy(out_v.at[b], out_hbm.at[pl.ds(off, P)], semo[b])

    def out_wait(b):
        pltpu.make_async_copy(out_v.at[b], out_hbm.at[pl.ds(0, P)],
                              semo[b]).wait()

    def compute(b):
        def group(g, c2):
            gs = pl.ds(g * 16, 16)
            wv = [w_v[b, k, gs] for k in range(4)]
            for lane in range(16):
                i = g * 16 + lane
                ws = [wv[k][lane] for k in range(4)]
                for j in range(C // 16):
                    s = pl.ds(j * 16, 16)
                    out_v[b, i, s] = (
                        q_v[b, 0, i, s] * ws[0]
                        + q_v[b, 0, P + i, s] * ws[1]
                        + q_v[b, 1, i, s] * ws[2]
                        + q_v[b, 1, P + i, s] * ws[3])
            return c2

        lax.fori_loop(0, P // 16, group, 0)

    idxw_start(0, 0)
    idxw_wait(0)
    gather_start(0)
    idxw_start(1, 1)

    def pair(itp, carry):
        for b in range(2):
            itb = itp * 2 + b

            @pl.when(itb < nit)
            def _():
                gather_wait(b)

                @pl.when(itb + 1 < nit)
                def _():
                    idxw_wait(1 - b)
                    gather_start(1 - b)

                @pl.when(itb >= 2)
                def _():
                    out_wait(b)

                compute(b)
                out_start(itb, b)

                @pl.when(itb + 2 < nit)
                def _():
                    idxw_start(itb + 2, b)
        return carry

    lax.fori_loop(0, (nit + 1) // 2, pair, 0)
    out_wait(0)
    out_wait(1)


@functools.cache
def _sc_gather():
    return pl.kernel(
        _sc_body,
        out_type=jax.ShapeDtypeStruct((NPAD, C), jnp.float32),
        mesh=plsc.VectorSubcoreMesh(core_axis_name="c", subcore_axis_name="s",
                                    num_cores=NC, num_subcores=NS),
        scratch_types=[
            pltpu.VMEM((2, 2, 2 * P), jnp.int32),
            pltpu.VMEM((2, 4, P), jnp.float32),
            pltpu.VMEM((2, 2, 2 * P, C), jnp.float32),
            pltpu.VMEM((2, P, C), jnp.float32),
            pltpu.SemaphoreType.DMA,
            pltpu.SemaphoreType.DMA,
            pltpu.SemaphoreType.DMA,
            pltpu.SemaphoreType.DMA,
            pltpu.SemaphoreType.DMA,
            pltpu.SemaphoreType.DMA,
        ],
    )


def kernel(img_features, points, R, T):
    table = jnp.transpose(img_features[0], (1, 2, 0)).reshape(H * H, C)
    p = points @ R + T
    xy = p[:, :2].T
    xy_pad = jnp.concatenate(
        [xy, jnp.broadcast_to(xy[:, :1], (2, NPAD - N))], axis=1)
    idx, w = _prelude(xy_pad)
    out_pad = _sc_gather()(table, idx, w)
    return out_pad[:N][None]


# speedup vs baseline: 1.0939x; 1.0939x over previous
import functools

import jax
import jax.numpy as jnp
import numpy as np
from jax import lax
from jax.experimental import pallas as pl
from jax.experimental.pallas import tpu as pltpu
from jax.experimental.pallas import tpu_sc as plsc

N = 100000
C = 128
H = 224
NC, NS = 2, 16
NW = NC * NS
NPW = 3200
NPAD = NW * NPW
P = 64
NPW0 = 63 * P
NPW1 = 37 * P


def _prelude_body(xy_ref, idx_ref, w_ref):
    size = float(H)
    x = xy_ref[0:1, :]
    y = xy_ref[1:2, :]
    x = x - jnp.min(x)
    x = x * ((size - 0.01) / jnp.max(x))
    y = y - jnp.min(y)
    y = y * ((size - 0.01) / jnp.max(y))
    x1 = jnp.floor(x)
    y1 = jnp.floor(y)
    x2 = jnp.minimum(jnp.ceil(x), size - 1.0)
    y2 = jnp.minimum(jnp.ceil(y), size - 1.0)
    wx1 = jnp.where(x2 > x1, x2 - x, 0.0)
    wx2 = jnp.where(x2 > x1, x - x1, 0.0)
    wy1 = jnp.where(y2 > y1, y2 - y, 0.0)
    wy2 = jnp.where(y2 > y1, y - y1, 0.0)
    x1i = x1.astype(jnp.int32)
    x2i = x2.astype(jnp.int32)
    y1i = y1.astype(jnp.int32)
    y2i = y2.astype(jnp.int32)
    idx_ref[0:1, :] = x1i * H + y1i
    idx_ref[1:2, :] = x1i * H + y2i
    idx_ref[2:3, :] = x2i * H + y1i
    idx_ref[3:4, :] = x2i * H + y2i
    w_ref[0:1, :] = wx1 * wy1
    w_ref[1:2, :] = wx1 * wy2
    w_ref[2:3, :] = wx2 * wy1
    w_ref[3:4, :] = wx2 * wy2


def _prelude(xy_pad):
    return pl.pallas_call(
        _prelude_body,
        out_shape=(
            jax.ShapeDtypeStruct((4, NPAD), jnp.int32),
            jax.ShapeDtypeStruct((4, NPAD), jnp.float32),
        ),
    )(xy_pad)


def _sc_body(table_hbm, idx_hbm, w_hbm, out_hbm, idx_v, w_v, q_v,
             out_v, semi0, semi1, semq0, semq1, semo0, semo1):
    cidx = lax.axis_index("c")
    sidx = lax.axis_index("s")
    base = jnp.where(cidx == 0, sidx * NPW0, NS * NPW0 + sidx * NPW1)
    nit = jnp.where(cidx == 0, NPW0 // P, NPW1 // P)
    semi = (semi0, semi1)
    semq = (semq0, semq1)
    semo = (semo0, semo1)

    def idxw_start(it, b):
        off = base + it * P
        for k in range(4):
            pltpu.async_copy(idx_hbm.at[k, pl.ds(off, P)],
                             idx_v.at[b, k // 2, pl.ds((k % 2) * P, P)],
                             semi[b])
            pltpu.async_copy(w_hbm.at[k, pl.ds(off, P)], w_v.at[b, k],
                             semi[b])

    def idxw_wait(b):
        for k in range(4):
            pltpu.make_async_copy(idx_hbm.at[k, pl.ds(0, P)],
                                  idx_v.at[b, k // 2, pl.ds((k % 2) * P, P)],
                                  semi[b]).wait()
            pltpu.make_async_copy(w_hbm.at[k, pl.ds(0, P)], w_v.at[b, k],
                                  semi[b]).wait()

    def gather_start(b):
        for j in range(2):
            pltpu.async_copy(table_hbm.at[idx_v.at[b, j]], q_v.at[b, j],
                             semq[b])

    def gather_wait(b):
        for j in range(2):
            pltpu.make_async_copy(table_hbm.at[idx_v.at[b, j]], q_v.at[b, j],
                                  semq[b]).wait()

    def out_start(it, b):
        off = base + it * P
        pltpu.async_copy(out_v.at[b], out_hbm.at[pl.ds(off, P)], semo[b])

    def out_wait(b):
        pltpu.make_async_copy(out_v.at[b], out_hbm.at[pl.ds(0, P)],
                              semo[b]).wait()

    def compute(b):
        def group(g, c2):
            gs = pl.ds(g * 16, 16)
            wv = [w_v[b, k, gs] for k in range(4)]
            for lane in range(16):
                i = g * 16 + lane
                ws = [wv[k][lane] for k in range(4)]
                for j in range(C // 16):
                    s = pl.ds(j * 16, 16)
                    out_v[b, i, s] = (
                        q_v[b, 0, i, s] * ws[0]
                        + q_v[b, 0, P + i, s] * ws[1]
                        + q_v[b, 1, i, s] * ws[2]
                        + q_v[b, 1, P + i, s] * ws[3])
            return c2

        lax.fori_loop(0, P // 16, group, 0)

    idxw_start(0, 0)
    idxw_wait(0)
    gather_start(0)
    idxw_start(1, 1)

    def pair(itp, carry):
        for b in range(2):
            itb = itp * 2 + b

            @pl.when(itb < nit)
            def _():
                gather_wait(b)

                @pl.when(itb + 1 < nit)
                def _():
                    idxw_wait(1 - b)
                    gather_start(1 - b)

                @pl.when(itb >= 2)
                def _():
                    out_wait(b)

                compute(b)
                out_start(itb, b)

                @pl.when(itb + 2 < nit)
                def _():
                    idxw_start(itb + 2, b)
        return carry

    lax.fori_loop(0, (nit + 1) // 2, pair, 0)
    out_wait(0)
    out_wait(1)


@functools.cache
def _sc_gather():
    return pl.kernel(
        _sc_body,
        out_type=jax.ShapeDtypeStruct((NPAD, C), jnp.float32),
        mesh=plsc.VectorSubcoreMesh(core_axis_name="c", subcore_axis_name="s",
                                    num_cores=NC, num_subcores=NS),
        scratch_types=[
            pltpu.VMEM((2, 2, 2 * P), jnp.int32),
            pltpu.VMEM((2, 4, P), jnp.float32),
            pltpu.VMEM((2, 2, 2 * P, C), jnp.float32),
            pltpu.VMEM((2, P, C), jnp.float32),
            pltpu.SemaphoreType.DMA,
            pltpu.SemaphoreType.DMA,
            pltpu.SemaphoreType.DMA,
            pltpu.SemaphoreType.DMA,
            pltpu.SemaphoreType.DMA,
            pltpu.SemaphoreType.DMA,
        ],
    )


def kernel(img_features, points, R, T):
    table = jnp.transpose(img_features[0], (1, 2, 0)).reshape(H * H, C)
    p = points @ R + T
    xy = p[:, :2].T
    xy_pad = jnp.concatenate(
        [xy, jnp.broadcast_to(xy[:, :1], (2, NPAD - N))], axis=1)
    idx, w = _prelude(xy_pad)
    out_pad = _sc_gather()(table, idx, w)
    return out_pad[:N][None]


# speedup vs baseline: 1.1680x; 1.0677x over previous
import functools

import jax
import jax.numpy as jnp
import numpy as np
from jax import lax
from jax.experimental import pallas as pl
from jax.experimental.pallas import tpu as pltpu
from jax.experimental.pallas import tpu_sc as plsc

N = 100000
C = 128
H = 224
NC, NS = 2, 16
NW = NC * NS
NPW = 3200
NPAD = NW * NPW
P = 64
NPW0 = 75 * P
NPW1 = 25 * P


def _prelude_body(xy_ref, idx_ref, w_ref):
    size = float(H)
    x = xy_ref[0:1, :]
    y = xy_ref[1:2, :]
    x = x - jnp.min(x)
    x = x * ((size - 0.01) / jnp.max(x))
    y = y - jnp.min(y)
    y = y * ((size - 0.01) / jnp.max(y))
    x1 = jnp.floor(x)
    y1 = jnp.floor(y)
    x2 = jnp.minimum(jnp.ceil(x), size - 1.0)
    y2 = jnp.minimum(jnp.ceil(y), size - 1.0)
    wx1 = jnp.where(x2 > x1, x2 - x, 0.0)
    wx2 = jnp.where(x2 > x1, x - x1, 0.0)
    wy1 = jnp.where(y2 > y1, y2 - y, 0.0)
    wy2 = jnp.where(y2 > y1, y - y1, 0.0)
    x1i = x1.astype(jnp.int32)
    x2i = x2.astype(jnp.int32)
    y1i = y1.astype(jnp.int32)
    y2i = y2.astype(jnp.int32)
    idx_ref[0:1, :] = x1i * H + y1i
    idx_ref[1:2, :] = x1i * H + y2i
    idx_ref[2:3, :] = x2i * H + y1i
    idx_ref[3:4, :] = x2i * H + y2i
    w_ref[0:1, :] = wx1 * wy1
    w_ref[1:2, :] = wx1 * wy2
    w_ref[2:3, :] = wx2 * wy1
    w_ref[3:4, :] = wx2 * wy2


def _prelude(xy_pad):
    return pl.pallas_call(
        _prelude_body,
        out_shape=(
            jax.ShapeDtypeStruct((4, NPAD), jnp.int32),
            jax.ShapeDtypeStruct((4, NPAD), jnp.float32),
        ),
    )(xy_pad)


def _sc_body(table_hbm, idx_hbm, w_hbm, out_hbm, idx_v, w_v, q_v,
             out_v, semi0, semi1, semq0, semq1, semo0, semo1):
    cidx = lax.axis_index("c")
    sidx = lax.axis_index("s")
    base = jnp.where(cidx == 0, sidx * NPW0, NS * NPW0 + sidx * NPW1)
    nit = jnp.where(cidx == 0, NPW0 // P, NPW1 // P)
    semi = (semi0, semi1)
    semq = (semq0, semq1)
    semo = (semo0, semo1)

    def idxw_start(it, b):
        off = base + it * P
        for k in range(4):
            pltpu.async_copy(idx_hbm.at[k, pl.ds(off, P)],
                             idx_v.at[b, k // 2, pl.ds((k % 2) * P, P)],
                             semi[b])
            pltpu.async_copy(w_hbm.at[k, pl.ds(off, P)], w_v.at[b, k],
                             semi[b])

    def idxw_wait(b):
        for k in range(4):
            pltpu.make_async_copy(idx_hbm.at[k, pl.ds(0, P)],
                                  idx_v.at[b, k // 2, pl.ds((k % 2) * P, P)],
                                  semi[b]).wait()
            pltpu.make_async_copy(w_hbm.at[k, pl.ds(0, P)], w_v.at[b, k],
                                  semi[b]).wait()

    def gather_start(b):
        for j in range(2):
            pltpu.async_copy(table_hbm.at[idx_v.at[b, j]], q_v.at[b, j],
                             semq[b])

    def gather_wait(b):
        for j in range(2):
            pltpu.make_async_copy(table_hbm.at[idx_v.at[b, j]], q_v.at[b, j],
                                  semq[b]).wait()

    def out_start(it, b):
        off = base + it * P
        pltpu.async_copy(out_v.at[b], out_hbm.at[pl.ds(off, P)], semo[b])

    def out_wait(b):
        pltpu.make_async_copy(out_v.at[b], out_hbm.at[pl.ds(0, P)],
                              semo[b]).wait()

    def compute(b):
        def group(g, c2):
            gs = pl.ds(g * 16, 16)
            wv = [w_v[b, k, gs] for k in range(4)]
            for lane in range(16):
                i = g * 16 + lane
                ws = [wv[k][lane] for k in range(4)]
                for j in range(C // 16):
                    s = pl.ds(j * 16, 16)
                    out_v[b, i, s] = (
                        q_v[b, 0, i, s] * ws[0]
                        + q_v[b, 0, P + i, s] * ws[1]
                        + q_v[b, 1, i, s] * ws[2]
                        + q_v[b, 1, P + i, s] * ws[3])
            return c2

        lax.fori_loop(0, P // 16, group, 0)

    idxw_start(0, 0)
    idxw_wait(0)
    gather_start(0)
    idxw_start(1, 1)

    def pair(itp, carry):
        for b in range(2):
            itb = itp * 2 + b

            @pl.when(itb < nit)
            def _():
                gather_wait(b)

                @pl.when(itb + 1 < nit)
                def _():
                    idxw_wait(1 - b)
                    gather_start(1 - b)

                @pl.when(itb >= 2)
                def _():
                    out_wait(b)

                compute(b)
                out_start(itb, b)

                @pl.when(itb + 2 < nit)
                def _():
                    idxw_start(itb + 2, b)
        return carry

    lax.fori_loop(0, (nit + 1) // 2, pair, 0)
    out_wait(0)
    out_wait(1)


@functools.cache
def _sc_gather():
    return pl.kernel(
        _sc_body,
        out_type=jax.ShapeDtypeStruct((NPAD, C), jnp.float32),
        mesh=plsc.VectorSubcoreMesh(core_axis_name="c", subcore_axis_name="s",
                                    num_cores=NC, num_subcores=NS),
        scratch_types=[
            pltpu.VMEM((2, 2, 2 * P), jnp.int32),
            pltpu.VMEM((2, 4, P), jnp.float32),
            pltpu.VMEM((2, 2, 2 * P, C), jnp.float32),
            pltpu.VMEM((2, P, C), jnp.float32),
            pltpu.SemaphoreType.DMA,
            pltpu.SemaphoreType.DMA,
            pltpu.SemaphoreType.DMA,
            pltpu.SemaphoreType.DMA,
            pltpu.SemaphoreType.DMA,
            pltpu.SemaphoreType.DMA,
        ],
    )


def kernel(img_features, points, R, T):
    table = jnp.transpose(img_features[0], (1, 2, 0)).reshape(H * H, C)
    p = points @ R + T
    xy = p[:, :2].T
    xy_pad = jnp.concatenate(
        [xy, jnp.broadcast_to(xy[:, :1], (2, NPAD - N))], axis=1)
    idx, w = _prelude(xy_pad)
    out_pad = _sc_gather()(table, idx, w)
    return out_pad[:N][None]
